# baseline (speedup 1.0000x reference)
import functools

import jax
import jax.numpy as jnp
from jax import lax
from jax.experimental import pallas as pl
from jax.experimental.pallas import tpu as pltpu
from jax.experimental.pallas import tpu_sc as plsc

PAD = 0
NC, NS, LANES = 2, 16, 16
NW = NC * NS
NSETS = 3


@jax.jit
def _embed(enc_flat, dec_flat, src_table, trg_table, pos_table):
    R = enc_flat.shape[0]
    V, H = src_table.shape
    L = 2048
    NB = R // L
    C = 16
    pos_per_w = L // NW
    NPC = pos_per_w // C
    NSUB = NB * NPC * 2
    NTOK = NB * 2 * pos_per_w

    mesh = plsc.VectorSubcoreMesh(core_axis_name="c", subcore_axis_name="s")

    scratch = [
        pltpu.VMEM((NTOK,), jnp.int32),
        pltpu.SemaphoreType.DMA,
    ]
    for _ in range(NSETS):
        scratch += [
            pltpu.VMEM((C, H), jnp.float32),
            pltpu.VMEM((C, H), jnp.float32),
            pltpu.VMEM((C,), jnp.int32),
            pltpu.SemaphoreType.DMA,
            pltpu.SemaphoreType.DMA,
            pltpu.SemaphoreType.DMA,
        ]

    @functools.partial(
        pl.kernel,
        out_type=(
            jax.ShapeDtypeStruct((R, H), jnp.float32),
            jax.ShapeDtypeStruct((R, H), jnp.float32),
        ),
        mesh=mesh,
        scratch_types=scratch,
    )
    def body(enc_hbm, dec_hbm, src_hbm, trg_hbm, pos_hbm,
             enc_out, dec_out, tokbuf, sem_tok, *bufs):
        sets = [bufs[i * 6:(i + 1) * 6] for i in range(NSETS)]
        toks = (enc_hbm, dec_hbm)
        tables = (src_hbm, trg_hbm)
        outs = (enc_out, dec_out)

        wid = lax.axis_index("s") * NC + lax.axis_index("c")
        l0 = wid * pos_per_w
        iot = lax.iota(jnp.int32, LANES)

        def segs(k):
            b, (pc, tbl) = k // 8, divmod(k % 8, 2)
            return b, pc, tbl, (b * 2 + tbl) * pos_per_w + pc * C

        for b in range(NB):
            for tbl in range(2):
                pltpu.async_copy(
                    toks[tbl].at[pl.ds(b * L + l0, pos_per_w)],
                    tokbuf.at[pl.ds((b * 2 + tbl) * pos_per_w, pos_per_w)],
                    sem_tok)
        for b in range(NB):
            for tbl in range(2):
                pltpu.make_async_copy(
                    toks[tbl].at[pl.ds(b * L + l0, pos_per_w)],
                    tokbuf.at[pl.ds((b * 2 + tbl) * pos_per_w, pos_per_w)],
                    sem_tok).wait()

        def issue(k):
            _, pc, tbl, off = segs(k)
            wr, pr, px, sw, sp, _ = sets[k % NSETS]
            t = tokbuf[pl.ds(off, C)]
            px[...] = jnp.where(t == PAD, 0, iot + (l0 + pc * C + 1))
            pltpu.async_copy(tables[tbl].at[tokbuf.at[pl.ds(off, C)]],
                             wr, sw)
            pltpu.async_copy(pos_hbm.at[px], pr, sp)

        def gather_drain(k):
            _, pc, tbl, off = segs(k)
            wr, pr, px, sw, sp, _ = sets[k % NSETS]
            pltpu.make_async_copy(tables[tbl].at[tokbuf.at[pl.ds(off, C)]],
                                  wr, sw).wait()
            pltpu.make_async_copy(pos_hbm.at[px], pr, sp).wait()

        def out_issue(k):
            b, pc, tbl, _ = segs(k)
            wr, _, _, _, _, so = sets[k % NSETS]
            base = b * L + l0 + pc * C
            pltpu.async_copy(wr, outs[tbl].at[pl.ds(base, C)], so)

        def out_drain(k):
            _, _, tbl, _ = segs(k)
            wr, _, _, _, _, so = sets[k % NSETS]
            pltpu.make_async_copy(wr, outs[tbl].at[pl.ds(0, C)], so).wait()

        def finish(k):
            wr, pr, _, _, _, _ = sets[k % NSETS]
            gather_drain(k)

            def add_col(j, _):
                for jj in range(2):
                    sl = pl.ds((2 * j + jj) * LANES, LANES)
                    for r in range(C):
                        plsc.addupdate(wr.at[r, sl], pr[r, sl])
                return 0

            lax.fori_loop(0, H // LANES // 2, add_col, 0)
            out_issue(k)

        issue(0)
        issue(1)
        for k in range(NSUB):
            if 1 <= k <= NSUB - 3:
                out_drain(k - 1)
            if k + 2 < NSUB:
                issue(k + 2)
            finish(k)
        out_drain(NSUB - 3)
        out_drain(NSUB - 2)
        out_drain(NSUB - 1)

    return body(enc_flat, dec_flat, src_table, trg_table, pos_table)


def kernel(encoder_inputs, decoder_inputs, src_table, trg_table, pos_table):
    B, L = encoder_inputs.shape
    H = src_table.shape[1]
    enc_flat = encoder_inputs.reshape(-1).astype(jnp.int32)
    dec_flat = decoder_inputs.reshape(-1).astype(jnp.int32)
    enc_out, dec_out = _embed(enc_flat, dec_flat, src_table, trg_table,
                              pos_table)
    return enc_out.reshape(B, L, H), dec_out.reshape(B, L, H)


# speedup vs baseline: 1.2715x; 1.2715x over previous
import functools

import jax
import jax.numpy as jnp
from jax import lax
from jax.experimental import pallas as pl
from jax.experimental.pallas import tpu as pltpu
from jax.experimental.pallas import tpu_sc as plsc

PAD = 0
NC, NS, LANES = 2, 16, 16
NW = NC * NS
NSETS = 3


@jax.jit
def _embed(enc_flat, dec_flat, src_table, trg_table, pos_table):
    R = enc_flat.shape[0]
    V, H = src_table.shape
    L = 2048
    NB = R // L
    C = 16
    pos_per_w = L // NW
    NPC = pos_per_w // C
    NSUB = NB * NPC * 2
    NTOK = NB * 2 * pos_per_w

    mesh = plsc.VectorSubcoreMesh(core_axis_name="c", subcore_axis_name="s")

    scratch = [
        pltpu.VMEM((NTOK,), jnp.int32),
        pltpu.SemaphoreType.DMA,
    ]
    for _ in range(NSETS):
        scratch += [
            pltpu.VMEM((C, H), jnp.float32),
            pltpu.VMEM((C, H), jnp.float32),
            pltpu.VMEM((C,), jnp.int32),
            pltpu.SemaphoreType.DMA,
            pltpu.SemaphoreType.DMA,
            pltpu.SemaphoreType.DMA,
        ]

    @functools.partial(
        pl.kernel,
        out_type=(
            jax.ShapeDtypeStruct((R, H), jnp.float32),
            jax.ShapeDtypeStruct((R, H), jnp.float32),
        ),
        mesh=mesh,
        scratch_types=scratch,
    )
    def body(enc_hbm, dec_hbm, src_hbm, trg_hbm, pos_hbm,
             enc_out, dec_out, tokbuf, sem_tok, *bufs):
        sets = [bufs[i * 6:(i + 1) * 6] for i in range(NSETS)]
        toks = (enc_hbm, dec_hbm)
        tables = (src_hbm, trg_hbm)
        outs = (enc_out, dec_out)

        wid = lax.axis_index("s") * NC + lax.axis_index("c")
        l0 = wid * pos_per_w
        iot = lax.iota(jnp.int32, LANES)

        def segs(k):
            b, (pc, tbl) = k // 8, divmod(k % 8, 2)
            return b, pc, tbl, (b * 2 + tbl) * pos_per_w + pc * C

        for b in range(NB):
            for tbl in range(2):
                pltpu.async_copy(
                    toks[tbl].at[pl.ds(b * L + l0, pos_per_w)],
                    tokbuf.at[pl.ds((b * 2 + tbl) * pos_per_w, pos_per_w)],
                    sem_tok)
        for b in range(NB):
            for tbl in range(2):
                pltpu.make_async_copy(
                    toks[tbl].at[pl.ds(b * L + l0, pos_per_w)],
                    tokbuf.at[pl.ds((b * 2 + tbl) * pos_per_w, pos_per_w)],
                    sem_tok).wait()

        def issue(k):
            _, pc, tbl, off = segs(k)
            wr, pr, px, sw, sp, _ = sets[k % NSETS]
            t = tokbuf[pl.ds(off, C)]
            px[...] = jnp.where(t == PAD, 0, iot + (l0 + pc * C + 1))
            pltpu.async_copy(tables[tbl].at[tokbuf.at[pl.ds(off, C)]],
                             wr, sw)
            pltpu.async_copy(pos_hbm.at[px], pr, sp)

        def gather_drain(k):
            _, pc, tbl, off = segs(k)
            wr, pr, px, sw, sp, _ = sets[k % NSETS]
            pltpu.make_async_copy(tables[tbl].at[tokbuf.at[pl.ds(off, C)]],
                                  wr, sw).wait()
            pltpu.make_async_copy(pos_hbm.at[px], pr, sp).wait()

        def out_issue(k):
            b, pc, tbl, _ = segs(k)
            wr, _, _, _, _, so = sets[k % NSETS]
            base = b * L + l0 + pc * C
            pltpu.async_copy(wr, outs[tbl].at[pl.ds(base, C)], so)

        def out_drain(k):
            _, _, tbl, _ = segs(k)
            wr, _, _, _, _, so = sets[k % NSETS]
            pltpu.make_async_copy(wr, outs[tbl].at[pl.ds(0, C)], so).wait()

        def finish(k):
            wr, pr, _, _, _, _ = sets[k % NSETS]
            gather_drain(k)

            def add_col(j, _):
                for r in range(C):
                    plsc.addupdate(wr.at[r, pl.ds(j * LANES, LANES)],
                                   pr[r, pl.ds(j * LANES, LANES)])
                return 0

            lax.fori_loop(0, H // LANES, add_col, 0)
            out_issue(k)

        issue(0)
        issue(1)
        for k in range(NSUB):
            finish(k)
            if k >= 1:
                out_drain(k - 1)
            if k + 2 < NSUB:
                issue(k + 2)
        out_drain(NSUB - 1)

    return body(enc_flat, dec_flat, src_table, trg_table, pos_table)


def kernel(encoder_inputs, decoder_inputs, src_table, trg_table, pos_table):
    B, L = encoder_inputs.shape
    H = src_table.shape[1]
    enc_flat = encoder_inputs.reshape(-1).astype(jnp.int32)
    dec_flat = decoder_inputs.reshape(-1).astype(jnp.int32)
    enc_out, dec_out = _embed(enc_flat, dec_flat, src_table, trg_table,
                              pos_table)
    return enc_out.reshape(B, L, H), dec_out.reshape(B, L, H)
